# Initial kernel scaffold; baseline (speedup 1.0000x reference)
import jax
import jax.numpy as jnp
from jax.experimental import pallas as pl


def kernel(node_states, edges, edge_weights, kernel, kernel_attention):
    raise NotImplementedError("write your pallas kernel here")


# speedup vs baseline: 5.2390x; 5.2390x over previous
import jax
import jax.numpy as jnp
from jax import lax
from jax.experimental import pallas as pl
from jax.experimental.pallas import tpu as pltpu
from jax.experimental.pallas import tpu_sc as plsc

N = 10000
D = 256
U = 256

NC = 2
NS = 16
NW = NC * NS
L = 16
NR = 64
NPW = 160
CH = 64
NEG_SLOPE = 0.01


def _tc_body(ns_ref, w_ref, a_ref, h_ref, pq_ref):
    h = jnp.dot(ns_ref[...], w_ref[...], preferred_element_type=jnp.float32)
    h_ref[...] = h
    pq_ref[...] = jnp.dot(h, a_ref[...], preferred_element_type=jnp.float32)


def _tc_transform(ns, w, apad):
    blk = 1000
    return pl.pallas_call(
        _tc_body,
        grid=(N // blk,),
        in_specs=[
            pl.BlockSpec((blk, D), lambda i: (i, 0)),
            pl.BlockSpec((D, U), lambda i: (0, 0)),
            pl.BlockSpec((U, 128), lambda i: (0, 0)),
        ],
        out_specs=[
            pl.BlockSpec((blk, U), lambda i: (i, 0)),
            pl.BlockSpec((blk, 128), lambda i: (i, 0)),
        ],
        out_shape=[
            jax.ShapeDtypeStruct((N, U), jnp.float32),
            jax.ShapeDtypeStruct((N, 128), jnp.float32),
        ],
    )(ns, w, apad)


def _sc_body(h_hbm, pa_hbm, qa_hbm, src_hbm, dst_hbm, ew_hbm, bnd_hbm,
             out_hbm,
             bnd_v, pa_v, qa_v, src_v, dst_v, ew_v, idxl_v, s_v,
             rows_v, acc_v, accs_v, sem):
    cid = lax.axis_index("c")
    sid = lax.axis_index("s")
    w = cid * NS + sid
    f32 = jnp.float32
    z = jnp.zeros((L,), f32)

    pltpu.sync_copy(bnd_hbm, bnd_v)
    pltpu.sync_copy(qa_hbm, qa_v)

    for p in range(2):
        ri = w * 2 + p
        nstart = ri * NPW

        def _zacc(r, _):
            for j in range(U // L):
                acc_v[r, pl.ds(j * L, L)] = z
            accs_v[r, :] = z
            return 0
        lax.fori_loop(0, NPW, _zacc, 0)

        pltpu.sync_copy(pa_hbm.at[pl.ds(nstart, NPW)], pa_v)

        rv0 = jnp.full((L,), ri, jnp.int32)
        lo = plsc.load_gather(bnd_v, [rv0])[0]
        hi = plsc.load_gather(bnd_v, [rv0 + 1])[0]
        base = (lo // 8) * 8
        nch = jnp.maximum((hi - base + CH - 1) // CH, 0)

        def _chunk(k, _):
            off = pl.multiple_of(base + k * CH, 8)
            pltpu.sync_copy(src_hbm.at[pl.ds(off, CH)], src_v)
            pltpu.sync_copy(dst_hbm.at[pl.ds(off, CH)], dst_v)
            pltpu.sync_copy(ew_hbm.at[pl.ds(off, CH)], ew_v)
            cp = pltpu.async_copy(h_hbm.at[dst_v], rows_v, sem)
            for g in range(CH // L):
                sl = pl.ds(g * L, L)
                srcg = src_v[sl]
                dstg = dst_v[sl]
                ewg = ew_v[sl]
                eidx = off + g * L + lax.iota(jnp.int32, L)
                inr = (eidx >= lo) & (eidx < hi)
                li = jnp.clip(srcg - nstart, 0, NPW - 1)
                pag = plsc.load_gather(pa_v, [li])
                qag = plsc.load_gather(qa_v, [dstg])
                x = ewg * (pag + qag)
                x = jnp.where(x >= 0.0, x, NEG_SLOPE * x)
                sc = jnp.exp(jnp.clip(x, -2.0, 2.0))
                sc = jnp.where(inr, sc, 0.0)
                s_v[sl] = sc
                idxl_v[sl] = li
            cp.wait()

            def _row(r, _):
                rv = jnp.full((L,), r, jnp.int32)
                li = plsc.load_gather(idxl_v, [rv])[0]
                svec = plsc.load_gather(s_v, [rv])
                plsc.addupdate(accs_v.at[li, :], svec)
                for j in range(U // L):
                    cs = pl.ds(j * L, L)
                    plsc.addupdate(acc_v.at[li, cs], rows_v[r, cs] * svec)
                return 0
            lax.fori_loop(0, CH, _row, 0)
            return 0
        lax.fori_loop(0, nch, _chunk, 0)

        def _norm(r, _):
            ivec = 1.0 / jnp.maximum(accs_v[r, :], 1e-30)
            for j in range(U // L):
                cs = pl.ds(j * L, L)
                acc_v[r, cs] = acc_v[r, cs] * ivec
            return 0
        lax.fori_loop(0, NPW, _norm, 0)

        nblocks = jnp.clip((N - nstart) // 16, 0, NPW // 16)

        def _out(bk, _):
            pltpu.sync_copy(acc_v.at[pl.ds(bk * 16, 16)],
                            out_hbm.at[pl.ds(nstart + bk * 16, 16)])
            return 0
        lax.fori_loop(0, nblocks, _out, 0)


def _sc_aggregate(h, pa, qa, src, dst, ew, bnd):
    f32 = jnp.float32
    mesh = plsc.VectorSubcoreMesh(
        core_axis_name="c", subcore_axis_name="s",
        num_cores=NC, num_subcores=NS)
    kfn = pl.kernel(
        _sc_body,
        out_type=jax.ShapeDtypeStruct((N, U), f32),
        mesh=mesh,
        compiler_params=pltpu.CompilerParams(needs_layout_passes=False),
        scratch_types=[
            pltpu.VMEM((80,), jnp.int32),
            pltpu.VMEM((NPW,), f32),
            pltpu.VMEM((NR * NPW,), f32),
            pltpu.VMEM((CH,), jnp.int32),
            pltpu.VMEM((CH,), jnp.int32),
            pltpu.VMEM((CH,), f32),
            pltpu.VMEM((CH,), jnp.int32),
            pltpu.VMEM((CH,), f32),
            pltpu.VMEM((CH, U), f32),
            pltpu.VMEM((NPW, U), f32),
            pltpu.VMEM((NPW, L), f32),
            pltpu.SemaphoreType.DMA,
        ],
    )
    return kfn(h, pa, qa, src, dst, ew, bnd)


def kernel(node_states, edges, edge_weights, kernel, kernel_attention):
    ns = node_states[0]
    e = edges[0]
    ew = edge_weights[0]
    n_edges = e.shape[0]

    att = kernel_attention[:, 0]
    apad = (jnp.zeros((U, 128), jnp.float32)
            .at[:, 0].set(att[:U]).at[:, 1].set(att[U:]))
    h, pq = _tc_transform(ns, kernel, apad)

    pa = jnp.zeros((NR * NPW,), jnp.float32).at[:N].set(pq[:, 0])
    qa = jnp.zeros((NR * NPW,), jnp.float32).at[:N].set(pq[:, 1])
    src = e[:, 0]
    dst = e[:, 1]
    srcp = jnp.pad(src, (0, CH))
    dstp = jnp.pad(dst, (0, CH))
    ewp = jnp.pad(ew, (0, CH))
    starts = jnp.arange(1, NR, dtype=jnp.int32) * NPW
    sp = jnp.searchsorted(src, starts).astype(jnp.int32)
    bnd = jnp.zeros((80,), jnp.int32).at[1:NR].set(sp).at[NR].set(n_edges)
    return _sc_aggregate(h, pa, qa, srcp, dstp, ewp, bnd)


# speedup vs baseline: 5.8606x; 1.1186x over previous
import jax
import jax.numpy as jnp
from jax import lax
from jax.experimental import pallas as pl
from jax.experimental.pallas import tpu as pltpu
from jax.experimental.pallas import tpu_sc as plsc

N = 10000
D = 256
U = 256

NC = 2
NS = 16
NW = NC * NS
L = 16
NR = 64
NPW = 160
CH = 64
NEG_SLOPE = 0.01


def _tc_body(ns_ref, w_ref, a_ref, h_ref, pq_ref):
    h = jnp.dot(ns_ref[...], w_ref[...], preferred_element_type=jnp.float32)
    h_ref[...] = h
    pq_ref[...] = jnp.dot(h, a_ref[...], preferred_element_type=jnp.float32)


def _tc_transform(ns, w, apad):
    blk = 1000
    return pl.pallas_call(
        _tc_body,
        grid=(N // blk,),
        in_specs=[
            pl.BlockSpec((blk, D), lambda i: (i, 0)),
            pl.BlockSpec((D, U), lambda i: (0, 0)),
            pl.BlockSpec((U, 128), lambda i: (0, 0)),
        ],
        out_specs=[
            pl.BlockSpec((blk, U), lambda i: (i, 0)),
            pl.BlockSpec((blk, 128), lambda i: (i, 0)),
        ],
        out_shape=[
            jax.ShapeDtypeStruct((N, U), jnp.float32),
            jax.ShapeDtypeStruct((N, 128), jnp.float32),
        ],
    )(ns, w, apad)


def _sc_body(h_hbm, pa_hbm, qa_hbm, src_hbm, dst_hbm, ew_hbm, bnd_hbm,
             out_hbm,
             bnd_v, pa_v, qa_v,
             src_a, dst_a, ew_a, idxl_a, s_a, rows_a,
             src_b, dst_b, ew_b, idxl_b, s_b, rows_b,
             acc_v, accs_v, sem_a, sem_b):
    cid = lax.axis_index("c")
    sid = lax.axis_index("s")
    w = cid * NS + sid
    f32 = jnp.float32
    z = jnp.zeros((L,), f32)

    pltpu.sync_copy(bnd_hbm, bnd_v)
    pltpu.sync_copy(qa_hbm, qa_v)

    def _load_edges(off, src_v, dst_v, ew_v):
        pltpu.sync_copy(src_hbm.at[pl.ds(off, CH)], src_v)
        pltpu.sync_copy(dst_hbm.at[pl.ds(off, CH)], dst_v)
        pltpu.sync_copy(ew_hbm.at[pl.ds(off, CH)], ew_v)

    for p in range(2):
        ri = w * 2 + p
        nstart = ri * NPW

        def _zacc(r, _):
            for j in range(U // L):
                acc_v[r, pl.ds(j * L, L)] = z
            accs_v[r, :] = z
            return 0
        lax.fori_loop(0, NPW, _zacc, 0)

        pltpu.sync_copy(pa_hbm.at[pl.ds(nstart, NPW)], pa_v)

        rv0 = jnp.full((L,), ri, jnp.int32)
        lo = plsc.load_gather(bnd_v, [rv0])[0]
        hi = plsc.load_gather(bnd_v, [rv0 + 1])[0]
        base = (lo // 8) * 8
        nch = jnp.maximum((hi - base + CH - 1) // CH, 0)

        def _scores(off, src_v, dst_v, ew_v, s_v, idxl_v):
            for g in range(CH // L):
                sl = pl.ds(g * L, L)
                srcg = src_v[sl]
                ewg = ew_v[sl]
                eidx = off + g * L + lax.iota(jnp.int32, L)
                inr = (eidx >= lo) & (eidx < hi)
                li = jnp.clip(srcg - nstart, 0, NPW - 1)
                pag = plsc.load_gather(pa_v, [li])
                qag = plsc.load_gather(qa_v, [dst_v[sl]])
                x = ewg * (pag + qag)
                x = jnp.where(x >= 0.0, x, NEG_SLOPE * x)
                sc = jnp.exp(jnp.clip(x, -2.0, 2.0))
                sc = jnp.where(inr, sc, 0.0)
                s_v[sl] = sc
                idxl_v[sl] = li

        def _accum(s_v, idxl_v, rows_v):
            def _row(r, _):
                rv = jnp.full((L,), r, jnp.int32)
                li = plsc.load_gather(idxl_v, [rv])[0]
                svec = plsc.load_gather(s_v, [rv])
                plsc.addupdate(accs_v.at[li, :], svec)
                for j in range(U // L):
                    cs = pl.ds(j * L, L)
                    plsc.addupdate(acc_v.at[li, cs], rows_v[r, cs] * svec)
                return 0
            lax.fori_loop(0, CH, _row, 0)

        nit = (nch + 1) // 2

        def _piter(k2, _):
            off_a = pl.multiple_of(base + (2 * k2) * CH, 8)
            off_b = pl.multiple_of(off_a + CH, 8)
            _load_edges(off_a, src_a, dst_a, ew_a)
            cp_a = pltpu.async_copy(h_hbm.at[dst_a], rows_a, sem_a)
            _scores(off_a, src_a, dst_a, ew_a, s_a, idxl_a)
            _load_edges(off_b, src_b, dst_b, ew_b)
            cp_b = pltpu.async_copy(h_hbm.at[dst_b], rows_b, sem_b)
            cp_a.wait()
            _accum(s_a, idxl_a, rows_a)
            _scores(off_b, src_b, dst_b, ew_b, s_b, idxl_b)
            cp_b.wait()
            _accum(s_b, idxl_b, rows_b)
            return 0
        lax.fori_loop(0, nit, _piter, 0)

        def _norm(r, _):
            ivec = 1.0 / jnp.maximum(accs_v[r, :], 1e-30)
            for j in range(U // L):
                cs = pl.ds(j * L, L)
                acc_v[r, cs] = acc_v[r, cs] * ivec
            return 0
        lax.fori_loop(0, NPW, _norm, 0)

        nblocks = jnp.clip((N - nstart) // 16, 0, NPW // 16)

        def _out(bk, _):
            pltpu.sync_copy(acc_v.at[pl.ds(bk * 16, 16)],
                            out_hbm.at[pl.ds(nstart + bk * 16, 16)])
            return 0
        lax.fori_loop(0, nblocks, _out, 0)


def _sc_aggregate(h, pa, qa, src, dst, ew, bnd):
    f32 = jnp.float32
    mesh = plsc.VectorSubcoreMesh(
        core_axis_name="c", subcore_axis_name="s",
        num_cores=NC, num_subcores=NS)
    kfn = pl.kernel(
        _sc_body,
        out_type=jax.ShapeDtypeStruct((N, U), f32),
        mesh=mesh,
        compiler_params=pltpu.CompilerParams(needs_layout_passes=False),
        scratch_types=[
            pltpu.VMEM((80,), jnp.int32),
            pltpu.VMEM((NPW,), f32),
            pltpu.VMEM((NR * NPW,), f32),
            pltpu.VMEM((CH,), jnp.int32),
            pltpu.VMEM((CH,), jnp.int32),
            pltpu.VMEM((CH,), f32),
            pltpu.VMEM((CH,), jnp.int32),
            pltpu.VMEM((CH,), f32),
            pltpu.VMEM((CH, U), f32),
            pltpu.VMEM((CH,), jnp.int32),
            pltpu.VMEM((CH,), jnp.int32),
            pltpu.VMEM((CH,), f32),
            pltpu.VMEM((CH,), jnp.int32),
            pltpu.VMEM((CH,), f32),
            pltpu.VMEM((CH, U), f32),
            pltpu.VMEM((NPW, U), f32),
            pltpu.VMEM((NPW, L), f32),
            pltpu.SemaphoreType.DMA,
            pltpu.SemaphoreType.DMA,
        ],
    )
    return kfn(h, pa, qa, src, dst, ew, bnd)


def kernel(node_states, edges, edge_weights, kernel, kernel_attention):
    ns = node_states[0]
    e = edges[0]
    ew = edge_weights[0]
    n_edges = e.shape[0]

    att = kernel_attention[:, 0]
    apad = (jnp.zeros((U, 128), jnp.float32)
            .at[:, 0].set(att[:U]).at[:, 1].set(att[U:]))
    h, pq = _tc_transform(ns, kernel, apad)

    pa = jnp.zeros((NR * NPW,), jnp.float32).at[:N].set(pq[:, 0])
    qa = jnp.zeros((NR * NPW,), jnp.float32).at[:N].set(pq[:, 1])
    src = e[:, 0]
    dst = e[:, 1]
    srcp = jnp.pad(src, (0, 4 * CH))
    dstp = jnp.pad(dst, (0, 4 * CH))
    ewp = jnp.pad(ew, (0, 4 * CH))
    starts = jnp.arange(1, NR, dtype=jnp.int32) * NPW
    sp = jnp.searchsorted(src, starts).astype(jnp.int32)
    bnd = jnp.zeros((80,), jnp.int32).at[1:NR].set(sp).at[NR].set(n_edges)
    return _sc_aggregate(h, pa, qa, srcp, dstp, ewp, bnd)
